# baseline (speedup 1.0000x reference)
import functools

import jax
import jax.numpy as jnp
from jax import lax
from jax.experimental import pallas as pl
from jax.experimental.pallas import tpu as pltpu
from jax.experimental.pallas import tpu_sc as plsc

_NC, _NS = 2, 16
_NW = _NC * _NS

_BATCH = 4096
_HIST = 200
_D = 32
_BB = _BATCH // _NW
_HC = 4
_NCHUNK = _HIST // _HC
_NBUF = 3
_K = 2

_mesh = plsc.VectorSubcoreMesh(core_axis_name="c", subcore_axis_name="s")


@functools.partial(
    pl.kernel,
    out_type=jax.ShapeDtypeStruct((_HIST, _D, _BATCH), jnp.float32),
    mesh=_mesh,
    scratch_types=(
        [pltpu.VMEM((_HIST, _BB), jnp.int32),
         pltpu.VMEM((_NBUF, _HC, _BB, _D), jnp.float32),
         pltpu.VMEM((_NBUF, _HC, _D, _BB + 1), jnp.float32)]
        + [pltpu.SemaphoreType.DMA] * (2 * _NBUF)
    ),
    compiler_params=pltpu.CompilerParams(use_tc_tiling_on_sc=False, needs_layout_passes=False),
)
def _gather_rows(xt_hbm, table_hbm, out_hbm, idx_all, rows, tpos, *sems):
    gsem = sems[:_NBUF]
    ssem = sems[_NBUF:]
    wid = lax.axis_index("s") * _NC + lax.axis_index("c")
    b0 = wid * _BB
    tbl = table_hbm
    iota16 = lax.iota(jnp.int32, 16)

    pltpu.sync_copy(xt_hbm.at[:, pl.ds(b0, _BB)], idx_all)

    def gather_start(c, b):
        for t in range(_HC):
            pltpu.make_async_copy(
                tbl.at[idx_all.at[c * _HC + t]],
                rows.at[b].at[t],
                gsem[b],
            ).start()

    def gather_wait(c, b):
        for t in range(_HC):
            pltpu.make_async_copy(
                tbl.at[idx_all.at[c * _HC + t]],
                rows.at[b].at[t],
                gsem[b],
            ).wait()

    def transpose_chunk(b):
        def blbody(bl16, carry):
            for t in range(_HC):
                src = rows.at[b].at[t]
                dst = tpos.at[b].at[t]
                for r in range(8):
                    bl = bl16 * 8 + r
                    blv = jnp.full((16,), 0, jnp.int32) + bl
                    v0 = src[bl, pl.ds(0, 16)]
                    v1 = src[bl, pl.ds(16, 16)]
                    plsc.store_scatter(dst, [iota16, blv], v0)
                    plsc.store_scatter(dst, [iota16 + 16, blv], v1)
            return carry
        lax.fori_loop(0, _BB // 8, blbody, 0)

    def store_desc(c, b):
        dst = out_hbm.at[pl.ds(c * _HC, _HC), pl.ds(0, _D), pl.ds(b0, _BB)]
        src = tpos.at[b].at[pl.ds(0, _HC), pl.ds(0, _D), pl.ds(0, _BB)]
        return pltpu.make_async_copy(src, dst, ssem[b])

    for c in range(_K):
        gather_start(c, c % _NBUF)

    for c in range(_NBUF - _K):
        gather_start(c + _K, (c + _K) % _NBUF)
        gather_wait(c, c % _NBUF)
        transpose_chunk(c % _NBUF)
        store_desc(c, c % _NBUF).start()

    def gen_body(g, carry):
        c0 = (_NBUF - _K) + g * _NBUF
        for t in range(_NBUF):
            c = c0 + t
            j = c + _K
            bj = t % _NBUF
            store_desc(j - _NBUF, bj).wait()
            gather_start(j, bj)
            b = (_NBUF - _K + t) % _NBUF
            gather_wait(c, b)
            transpose_chunk(b)
            store_desc(c, b).start()
        return carry

    ngen = (_NCHUNK - _NBUF) // _NBUF
    lax.fori_loop(0, ngen, gen_body, 0)

    done = (_NBUF - _K) + ngen * _NBUF
    for c in range(done, _NCHUNK - _K):
        j = c + _K
        store_desc(j - _NBUF, j % _NBUF).wait()
        gather_start(j, j % _NBUF)
        gather_wait(c, c % _NBUF)
        transpose_chunk(c % _NBUF)
        store_desc(c, c % _NBUF).start()

    for c in range(_NCHUNK - _K, _NCHUNK):
        gather_wait(c, c % _NBUF)
        transpose_chunk(c % _NBUF)
        store_desc(c, c % _NBUF).start()

    for c in range(_NCHUNK - _NBUF, _NCHUNK):
        store_desc(c, c % _NBUF).wait()


def kernel(X, position_table):
    xt = (X.T * 4).astype(jnp.int32)
    pt_pad = jnp.concatenate(
        [position_table, jnp.zeros((1000000, 96), jnp.float32)], axis=1)
    tbl4 = jnp.reshape(pt_pad, (4000000, _D))
    out3 = _gather_rows(xt, tbl4)
    return out3.transpose(2, 0, 1)


# speedup vs baseline: 1.1702x; 1.1702x over previous
import functools

import jax
import jax.numpy as jnp
from jax import lax
from jax.experimental import pallas as pl
from jax.experimental.pallas import tpu as pltpu
from jax.experimental.pallas import tpu_sc as plsc

_NC, _NS = 2, 16
_NW = _NC * _NS

_BATCH = 4096
_HIST = 200
_D = 32
_BB = _BATCH // _NW
_HC = 4
_NCHUNK = _HIST // _HC
_NBUF = 3
_K = 2

_mesh = plsc.VectorSubcoreMesh(core_axis_name="c", subcore_axis_name="s")


@functools.partial(
    pl.kernel,
    out_type=jax.ShapeDtypeStruct((_HIST, _D // 8, _NW, 8, _BB), jnp.float32),
    mesh=_mesh,
    scratch_types=(
        [pltpu.VMEM((_HIST, _BB), jnp.int32),
         pltpu.VMEM((_NBUF, _HC, _BB, _D), jnp.float32),
         pltpu.VMEM((_NBUF, _HC, _D // 8, 1, 8, _BB + 1), jnp.float32)]
        + [pltpu.SemaphoreType.DMA] * (2 * _NBUF)
    ),
    compiler_params=pltpu.CompilerParams(use_tc_tiling_on_sc=False, needs_layout_passes=False),
)
def _gather_rows(xt_hbm, table_hbm, out_hbm, idx_all, rows, tpos, *sems):
    gsem = sems[:_NBUF]
    ssem = sems[_NBUF:]
    wid = lax.axis_index("s") * _NC + lax.axis_index("c")
    b0 = wid * _BB
    tbl = table_hbm
    iota16 = lax.iota(jnp.int32, 16)

    pltpu.sync_copy(xt_hbm.at[:, pl.ds(b0, _BB)], idx_all)

    def gather_start(c, b):
        for t in range(_HC):
            pltpu.make_async_copy(
                tbl.at[idx_all.at[c * _HC + t]],
                rows.at[b].at[t],
                gsem[b],
            ).start()

    def gather_wait(c, b):
        for t in range(_HC):
            pltpu.make_async_copy(
                tbl.at[idx_all.at[c * _HC + t]],
                rows.at[b].at[t],
                gsem[b],
            ).wait()

    tr_lo = iota16 // 8
    ds_all = iota16 % 8
    zero16 = jnp.zeros((16,), jnp.int32)

    def transpose_chunk(b):
        def blbody(bl16, carry):
            for t in range(_HC):
                src = rows.at[b].at[t]
                dst = tpos.at[b].at[t]
                for r in range(8):
                    bl = bl16 * 8 + r
                    blv = jnp.full((16,), 0, jnp.int32) + bl
                    v0 = src[bl, pl.ds(0, 16)]
                    v1 = src[bl, pl.ds(16, 16)]
                    plsc.store_scatter(dst, [tr_lo, zero16, ds_all, blv], v0)
                    plsc.store_scatter(dst, [tr_lo + 2, zero16, ds_all, blv],
                                       v1)
            return carry
        lax.fori_loop(0, _BB // 8, blbody, 0)

    def store_desc(c, b):
        dst = out_hbm.at[pl.ds(c * _HC, _HC), pl.ds(0, _D // 8),
                         pl.ds(wid, 1)]
        src = tpos.at[b].at[pl.ds(0, _HC), pl.ds(0, _D // 8), pl.ds(0, 1),
                            pl.ds(0, 8), pl.ds(0, _BB)]
        return pltpu.make_async_copy(src, dst, ssem[b])

    for c in range(_K):
        gather_start(c, c % _NBUF)

    for c in range(_NBUF - _K):
        gather_start(c + _K, (c + _K) % _NBUF)
        gather_wait(c, c % _NBUF)
        transpose_chunk(c % _NBUF)
        store_desc(c, c % _NBUF).start()

    def gen_body(g, carry):
        c0 = (_NBUF - _K) + g * _NBUF
        for t in range(_NBUF):
            c = c0 + t
            j = c + _K
            bj = t % _NBUF
            store_desc(j - _NBUF, bj).wait()
            gather_start(j, bj)
            b = (_NBUF - _K + t) % _NBUF
            gather_wait(c, b)
            transpose_chunk(b)
            store_desc(c, b).start()
        return carry

    ngen = (_NCHUNK - _NBUF) // _NBUF
    lax.fori_loop(0, ngen, gen_body, 0)

    done = (_NBUF - _K) + ngen * _NBUF
    for c in range(done, _NCHUNK - _K):
        j = c + _K
        store_desc(j - _NBUF, j % _NBUF).wait()
        gather_start(j, j % _NBUF)
        gather_wait(c, c % _NBUF)
        transpose_chunk(c % _NBUF)
        store_desc(c, c % _NBUF).start()

    for c in range(_NCHUNK - _K, _NCHUNK):
        gather_wait(c, c % _NBUF)
        transpose_chunk(c % _NBUF)
        store_desc(c, c % _NBUF).start()

    for c in range(_NCHUNK - _NBUF, _NCHUNK):
        store_desc(c, c % _NBUF).wait()


def kernel(X, position_table):
    xt = (X.T * 4).astype(jnp.int32)
    pt_pad = jnp.concatenate(
        [position_table, jnp.zeros((1000000, 96), jnp.float32)], axis=1)
    tbl4 = jnp.reshape(pt_pad, (4000000, _D))
    out5 = _gather_rows(xt, tbl4)
    return out5.transpose(2, 4, 0, 1, 3).reshape(_BATCH, _HIST, _D)


# speedup vs baseline: 1.1720x; 1.0015x over previous
import functools

import jax
import jax.numpy as jnp
from jax import lax
from jax.experimental import pallas as pl
from jax.experimental.pallas import tpu as pltpu
from jax.experimental.pallas import tpu_sc as plsc

_NC, _NS = 2, 16
_NW = _NC * _NS

_BATCH = 4096
_HIST = 200
_D = 32
_BB = _BATCH // _NW
_HC = 4
_NCHUNK = _HIST // _HC
_NBUF = 3
_K = 2

_mesh = plsc.VectorSubcoreMesh(core_axis_name="c", subcore_axis_name="s")


@functools.partial(
    pl.kernel,
    out_type=jax.ShapeDtypeStruct((_HIST, _D // 8, _NW, 8, _BB), jnp.float32),
    mesh=_mesh,
    scratch_types=(
        [pltpu.VMEM((_HIST, _BB), jnp.int32),
         pltpu.VMEM((_NBUF, _HC, _BB, _D), jnp.float32),
         pltpu.VMEM((_NBUF, _HC, _D // 8, 1, 8, _BB + 1), jnp.float32)]
        + [pltpu.SemaphoreType.DMA] * (2 * _NBUF)
    ),
    compiler_params=pltpu.CompilerParams(use_tc_tiling_on_sc=False, needs_layout_passes=False),
)
def _gather_rows(xt_hbm, table_hbm, out_hbm, idx_all, rows, tpos, *sems):
    gsem = sems[:_NBUF]
    ssem = sems[_NBUF:]
    wid = lax.axis_index("s") * _NC + lax.axis_index("c")
    b0 = wid * _BB
    tbl = table_hbm
    iota16 = lax.iota(jnp.int32, 16)

    pltpu.sync_copy(xt_hbm.at[:, pl.ds(b0, _BB)], idx_all)

    def gather_start(c, b):
        for t in range(_HC):
            pltpu.make_async_copy(
                tbl.at[idx_all.at[c * _HC + t]],
                rows.at[b].at[t],
                gsem[b],
            ).start()

    def gather_wait(c, b):
        for t in range(_HC):
            pltpu.make_async_copy(
                tbl.at[idx_all.at[c * _HC + t]],
                rows.at[b].at[t],
                gsem[b],
            ).wait()

    tr_lo = iota16 // 8
    ds_all = iota16 % 8
    zero16 = jnp.zeros((16,), jnp.int32)

    def transpose_chunk(b):
        def blbody(bl16, carry):
            for t in range(_HC):
                src = rows.at[b].at[t]
                dst = tpos.at[b].at[t]
                for r in range(8):
                    bl = bl16 * 8 + r
                    blv = jnp.full((16,), 0, jnp.int32) + bl
                    v0 = src[bl, pl.ds(0, 16)]
                    v1 = src[bl, pl.ds(16, 16)]
                    plsc.store_scatter(dst, [tr_lo, zero16, ds_all, blv], v0)
                    plsc.store_scatter(dst, [tr_lo + 2, zero16, ds_all, blv],
                                       v1)
            return carry
        lax.fori_loop(0, _BB // 8, blbody, 0)

    def store_desc(c, b):
        dst = out_hbm.at[pl.ds(c * _HC, _HC), pl.ds(0, _D // 8),
                         pl.ds(wid, 1)]
        src = tpos.at[b].at[pl.ds(0, _HC), pl.ds(0, _D // 8), pl.ds(0, 1),
                            pl.ds(0, 8), pl.ds(0, _BB)]
        return pltpu.make_async_copy(src, dst, ssem[b])

    for c in range(_K):
        gather_start(c, c % _NBUF)

    for c in range(_NBUF - _K):
        gather_start(c + _K, (c + _K) % _NBUF)
        gather_wait(c, c % _NBUF)
        transpose_chunk(c % _NBUF)
        store_desc(c, c % _NBUF).start()

    def gen_body(g, carry):
        c0 = (_NBUF - _K) + g * _NBUF
        for t in range(_NBUF):
            c = c0 + t
            j = c + _K
            bj = t % _NBUF
            store_desc(j - _NBUF, bj).wait()
            gather_start(j, bj)
            b = (_NBUF - _K + t) % _NBUF
            gather_wait(c, b)
            transpose_chunk(b)
            store_desc(c, b).start()
        return carry

    ngen = (_NCHUNK - _NBUF) // _NBUF
    lax.fori_loop(0, ngen, gen_body, 0)

    done = (_NBUF - _K) + ngen * _NBUF
    for c in range(done, _NCHUNK - _K):
        j = c + _K
        store_desc(j - _NBUF, j % _NBUF).wait()
        gather_start(j, j % _NBUF)
        gather_wait(c, c % _NBUF)
        transpose_chunk(c % _NBUF)
        store_desc(c, c % _NBUF).start()

    for c in range(_NCHUNK - _K, _NCHUNK):
        gather_wait(c, c % _NBUF)
        transpose_chunk(c % _NBUF)
        store_desc(c, c % _NBUF).start()

    for c in range(_NCHUNK - _NBUF, _NCHUNK):
        store_desc(c, c % _NBUF).wait()


def kernel(X, position_table):
    xt = (X.T * 4).astype(jnp.int32)
    pt_pad = jnp.pad(position_table, ((0, 0), (0, 96)))
    tbl4 = jnp.reshape(pt_pad, (4000000, _D))
    out5 = _gather_rows(xt, tbl4)
    return out5.transpose(2, 4, 0, 1, 3).reshape(_BATCH, _HIST, _D)
